# baseline (speedup 1.0000x reference)
import functools

import jax
import jax.numpy as jnp
from jax import lax
from jax.experimental import pallas as pl
from jax.experimental.pallas import tpu as pltpu
from jax.experimental.pallas import tpu_sc as plsc

L = 16
KK = 128
KV = KK // L
ROW = 8192
NROWS = 128 * 32
GROUPS = ROW // KK
NC = 2
NS = 16
NW = NC * NS
RPW = NROWS // NW


def _vsort_desc(v):
    k, _ = plsc.sort_key_val(v, v, descending=True)
    return k


def _vrev(v):
    return lax.rev(v, (0,))


def _bitonic_finish(C):
    n = len(C)
    d = n // 2
    while d >= 1:
        for s in range(0, n, 2 * d):
            for i in range(s, s + d):
                a, b = C[i], C[i + d]
                C[i], C[i + d] = jnp.maximum(a, b), jnp.minimum(a, b)
        d //= 2
    return [_vsort_desc(c) for c in C]


def _merge_full(A, B):
    C = list(A) + [_vrev(b) for b in reversed(B)]
    return _bitonic_finish(C)


def _merge_capped(A, B):
    RB = [_vrev(b) for b in reversed(B)]
    C = [jnp.maximum(a, rb) for a, rb in zip(A, RB)]
    return _bitonic_finish(C)


def _sc_topk(x_hbm, out_hbm, row_v, wa, wb, outs):
    wid = lax.axis_index("s") * NC + lax.axis_index("c")
    base = wid * RPW

    def row_body(j, carry):
        pltpu.sync_copy(x_hbm.at[base + j], row_v)

        @plsc.parallel_loop(0, GROUPS, 1, unroll=4)
        def group_body(g):
            runs = [[_vsort_desc(row_v[pl.ds(g * KK + k * L, L)])]
                    for k in range(KV)]
            while len(runs) > 1:
                runs = [_merge_full(runs[2 * t], runs[2 * t + 1])
                        for t in range(len(runs) // 2)]
            for k in range(KV):
                wa[pl.ds(g * KK + k * L, L)] = runs[0][k]

        cur, nxt = wa, wb
        for rnd in range(6):
            n_out = GROUPS >> (rnd + 1)

            def _make_cap_body(cur, nxt):
                def cap_body(i):
                    A = [cur[pl.ds((2 * i) * KK + k * L, L)]
                         for k in range(KV)]
                    B = [cur[pl.ds((2 * i + 1) * KK + k * L, L)]
                         for k in range(KV)]
                    R = _merge_capped(A, B)
                    for k in range(KV):
                        nxt[pl.ds(i * KK + k * L, L)] = R[k]
                return cap_body

            plsc.parallel_loop(0, n_out, 1, unroll=min(4, n_out))(
                _make_cap_body(cur, nxt))
            cur, nxt = nxt, cur

        for k in range(KV):
            outs[j, pl.ds(k * L, L)] = cur[pl.ds(k * L, L)]
        return carry

    lax.fori_loop(0, RPW, row_body, 0)
    pltpu.sync_copy(outs, out_hbm.at[pl.ds(base, RPW)])


_mesh = plsc.VectorSubcoreMesh(
    core_axis_name="c", subcore_axis_name="s", num_cores=NC, num_subcores=NS)

_topk_call = functools.partial(
    pl.kernel,
    out_type=jax.ShapeDtypeStruct((NROWS, KK), jnp.float32),
    mesh=_mesh,
    compiler_params=pltpu.CompilerParams(needs_layout_passes=False),
    scratch_types=[
        pltpu.VMEM((ROW,), jnp.float32),
        pltpu.VMEM((ROW,), jnp.float32),
        pltpu.VMEM((ROW,), jnp.float32),
        pltpu.VMEM((RPW, KK), jnp.float32),
    ],
)(_sc_topk)


@jax.jit
def kernel(input):
    x = input.reshape(NROWS, ROW)
    out = _topk_call(x)
    return out.reshape(128, 32, KK)


# speedup vs baseline: 1.3132x; 1.3132x over previous
import functools

import jax
import jax.numpy as jnp
from jax import lax
from jax.experimental import pallas as pl
from jax.experimental.pallas import tpu as pltpu
from jax.experimental.pallas import tpu_sc as plsc

L = 16
KK = 128
KV = KK // L
ROW = 8192
NROWS = 128 * 32
GROUPS = ROW // KK
NC = 2
NS = 16
NW = NC * NS
RPW = NROWS // NW


def _vsort_desc(v):
    k, _ = plsc.sort_key_val(v, v, descending=True)
    return k


def _vrev(v):
    return lax.rev(v, (0,))


def _bitonic_finish(C):
    n = len(C)
    d = n // 2
    while d >= 1:
        for s in range(0, n, 2 * d):
            for i in range(s, s + d):
                a, b = C[i], C[i + d]
                C[i], C[i + d] = jnp.maximum(a, b), jnp.minimum(a, b)
        d //= 2
    return [_vsort_desc(c) for c in C]


def _merge_full(A, B):
    C = list(A) + [_vrev(b) for b in reversed(B)]
    return _bitonic_finish(C)


def _merge_capped(A, B):
    RB = [_vrev(b) for b in reversed(B)]
    C = [jnp.maximum(a, rb) for a, rb in zip(A, RB)]
    return _bitonic_finish(C)


def _sc_topk(x_hbm, out_hbm, row_a, row_b, wa, wb, outs, sem_a, sem_b):
    wid = lax.axis_index("s") * NC + lax.axis_index("c")
    base = wid * RPW

    def process_row(row_v, j):
        @plsc.parallel_loop(0, GROUPS, 1, unroll=2)
        def group_body(g):
            runs = [[_vsort_desc(row_v[pl.ds(g * KK + k * L, L)])]
                    for k in range(KV)]
            while len(runs) > 1:
                runs = [_merge_full(runs[2 * t], runs[2 * t + 1])
                        for t in range(len(runs) // 2)]
            for k in range(KV):
                wa[pl.ds(g * KK + k * L, L)] = runs[0][k]

        cur, nxt = wa, wb
        for rnd in range(6):
            n_out = GROUPS >> (rnd + 1)

            def _make_cap_body(cur, nxt):
                def cap_body(i):
                    A = [cur[pl.ds((2 * i) * KK + k * L, L)]
                         for k in range(KV)]
                    B = [cur[pl.ds((2 * i + 1) * KK + k * L, L)]
                         for k in range(KV)]
                    R = _merge_capped(A, B)
                    for k in range(KV):
                        nxt[pl.ds(i * KK + k * L, L)] = R[k]
                return cap_body

            plsc.parallel_loop(0, n_out, 1, unroll=min(2, n_out))(
                _make_cap_body(cur, nxt))
            cur, nxt = nxt, cur

        for k in range(KV):
            outs[j, pl.ds(k * L, L)] = cur[pl.ds(k * L, L)]

    pltpu.make_async_copy(x_hbm.at[base], row_a, sem_a).start()

    def pair_body(p, carry):
        r0 = base + 2 * p
        pltpu.make_async_copy(x_hbm.at[r0 + 1], row_b, sem_b).start()
        pltpu.make_async_copy(x_hbm.at[r0], row_a, sem_a).wait()
        process_row(row_a, 2 * p)
        nxt = base + ((2 * p + 2) & (RPW - 1))
        pltpu.make_async_copy(x_hbm.at[nxt], row_a, sem_a).start()
        pltpu.make_async_copy(x_hbm.at[r0 + 1], row_b, sem_b).wait()
        process_row(row_b, 2 * p + 1)
        return carry

    lax.fori_loop(0, RPW // 2, pair_body, 0)
    pltpu.make_async_copy(x_hbm.at[base], row_a, sem_a).wait()
    pltpu.sync_copy(outs, out_hbm.at[pl.ds(base, RPW)])


_mesh = plsc.VectorSubcoreMesh(
    core_axis_name="c", subcore_axis_name="s", num_cores=NC, num_subcores=NS)

_topk_call = functools.partial(
    pl.kernel,
    out_type=jax.ShapeDtypeStruct((NROWS, KK), jnp.float32),
    mesh=_mesh,
    compiler_params=pltpu.CompilerParams(needs_layout_passes=False),
    scratch_types=[
        pltpu.VMEM((ROW,), jnp.float32),
        pltpu.VMEM((ROW,), jnp.float32),
        pltpu.VMEM((ROW,), jnp.float32),
        pltpu.VMEM((ROW,), jnp.float32),
        pltpu.VMEM((RPW, KK), jnp.float32),
        pltpu.SemaphoreType.DMA,
        pltpu.SemaphoreType.DMA,
    ],
)(_sc_topk)


@jax.jit
def kernel(input):
    x = input.reshape(NROWS, ROW)
    out = _topk_call(x)
    return out.reshape(128, 32, KK)
